# Initial kernel scaffold; baseline (speedup 1.0000x reference)
import jax
import jax.numpy as jnp
from jax.experimental import pallas as pl


def kernel(scores, target):
    raise NotImplementedError("write your pallas kernel here")


# speedup vs baseline: 6.6584x; 6.6584x over previous
import functools

import jax
import jax.numpy as jnp
import numpy as np
from jax import lax
from jax.experimental import pallas as pl
from jax.experimental.pallas import tpu as pltpu
from jax.experimental.pallas import tpu_sc as plsc

M = 16384
S = 5
NUM_TILES = 32
CHUNK = M // NUM_TILES
VPC = CHUNK // 16
MARGIN = 1.0

_mesh = plsc.VectorSubcoreMesh(core_axis_name="c", subcore_axis_name="s")


def _randint_bits():
    k1, k2 = jax.random.split(jax.random.key(42))
    hb = jax.random.bits(k1, (M, S), jnp.uint32).T.reshape(-1)
    lb = jax.random.bits(k2, (M, S), jnp.uint32).T.reshape(-1)
    return hb, lb


def _compact_body(scores_hbm, target_hbm, c_hbm, p_hbm, tgt_v, sc_v, idx_v, pv):
    wid = lax.axis_index("c") * 16 + lax.axis_index("s")
    base = wid * CHUNK

    pltpu.sync_copy(target_hbm, tgt_v)
    for q in range(4):
        pltpu.sync_copy(scores_hbm.at[pl.ds(base + q * 128, 128)], sc_v.at[q])

    def pref_body(i, a):
        return a + tgt_v[pl.ds(i * 16, 16)]

    accv = lax.fori_loop(0, wid * VPC, pref_body, jnp.zeros((16,), jnp.int32))
    pos_before = jnp.sum(accv)

    iot = lax.iota(jnp.int32, 16)
    carry = jnp.int32(0)
    for k in range(VPC):
        t = tgt_v[pl.ds(base + k * 16, 16)]
        csum = plsc.cumsum(t)
        excl = csum - t
        prank = pos_before + carry + excl
        gidx = base + k * 16 + iot
        dest = jnp.where(t == 1, prank, M + gidx - prank)
        idx_v[k // 8, pl.ds((k % 8) * 16, 16)] = dest
        carry = carry + jnp.sum(t)

    for q in range(4):
        pltpu.sync_copy(sc_v.at[q], c_hbm.at[idx_v.at[q]])

    @pl.when(wid == NUM_TILES - 1)
    def _():
        pv[...] = jnp.broadcast_to(pos_before + carry, (16,))
        pltpu.sync_copy(pv, p_hbm)


def _sample_body(c_hbm, p_hbm, hb_hbm, lb_hbm, out_hbm,
                 negv, posv, hbv, lbv, pvv, accv):
    wid = lax.axis_index("c") * 16 + lax.axis_index("s")
    base = wid * CHUNK

    pltpu.sync_copy(c_hbm.at[pl.ds(M, M)], negv)
    pltpu.sync_copy(c_hbm.at[pl.ds(base, CHUNK)], posv)
    for s in range(S):
        pltpu.sync_copy(hb_hbm.at[pl.ds(s * M + base, CHUNK)],
                        hbv.at[pl.ds(s * CHUNK, CHUNK)])
        pltpu.sync_copy(lb_hbm.at[pl.ds(s * M + base, CHUNK)],
                        lbv.at[pl.ds(s * CHUNK, CHUNK)])
    pltpu.sync_copy(p_hbm, pvv)

    P = pvv[...]
    span = jnp.maximum(M - P, 1).astype(jnp.uint32)
    m1 = lax.rem(jnp.full((16,), 65536, jnp.uint32), span)
    m2 = lax.rem(m1 * m1, span)
    iot = lax.iota(jnp.int32, 16)

    def body(i, acc):
        s = i // VPC
        k = i - s * VPC
        off = s * CHUNK + k * 16
        hb = hbv[pl.ds(off, 16)]
        lb = lbv[pl.ds(off, 16)]
        r = lax.rem(hb, span) * m2 + lax.rem(lb, span)
        samp = plsc.bitcast(lax.rem(r, span), jnp.int32)
        neg = plsc.load_gather(negv, [samp])
        p = posv[pl.ds(k * 16, 16)]
        rowid = base + k * 16 + iot
        term = jnp.maximum(MARGIN - p + neg, 0.0)
        return acc + jnp.where(rowid < P, term, 0.0)

    acc = lax.fori_loop(0, S * VPC, body, jnp.zeros((16,), jnp.float32))
    accv[...] = acc
    pltpu.sync_copy(accv, out_hbm.at[pl.ds(wid * 16, 16)])


_compact = pl.kernel(
    _compact_body,
    out_type=[jax.ShapeDtypeStruct((2 * M,), jnp.float32),
              jax.ShapeDtypeStruct((16,), jnp.int32)],
    mesh=_mesh,
    compiler_params=pltpu.CompilerParams(needs_layout_passes=False),
    scratch_types=[
        pltpu.VMEM((M,), jnp.int32),
        pltpu.VMEM((4, 128), jnp.float32),
        pltpu.VMEM((4, 128), jnp.int32),
        pltpu.VMEM((16,), jnp.int32),
    ],
)

_sample = pl.kernel(
    _sample_body,
    out_type=jax.ShapeDtypeStruct((NUM_TILES * 16,), jnp.float32),
    mesh=_mesh,
    compiler_params=pltpu.CompilerParams(needs_layout_passes=False),
    scratch_types=[
        pltpu.VMEM((M,), jnp.float32),
        pltpu.VMEM((CHUNK,), jnp.float32),
        pltpu.VMEM((S * CHUNK,), jnp.uint32),
        pltpu.VMEM((S * CHUNK,), jnp.uint32),
        pltpu.VMEM((16,), jnp.int32),
        pltpu.VMEM((16,), jnp.float32),
    ],
)


def kernel(scores, target):
    hb, lb = _randint_bits()
    c, pvec = _compact(scores, target)
    parts = _sample(c, pvec, hb, lb)
    P = pvec[0]
    return parts.sum() / (P * S)


# speedup vs baseline: 7.5228x; 1.1298x over previous
import jax
import jax.numpy as jnp
import numpy as np
from jax import lax
from jax.experimental import pallas as pl
from jax.experimental.pallas import tpu as pltpu
from jax.experimental.pallas import tpu_sc as plsc

M = 16384
S = 5
NUM_TILES = 32
CHUNK = M // NUM_TILES
VPC = CHUNK // 16
BITS_PER_TILE = 2 * S * CHUNK
MARGIN = 1.0

_mesh = plsc.VectorSubcoreMesh(core_axis_name="c", subcore_axis_name="s")


def _randint_bits():
    k1, k2 = jax.random.split(jax.random.key(42))
    hb = jax.random.bits(k1, (M, S), jnp.uint32).T.reshape(S, NUM_TILES, CHUNK)
    lb = jax.random.bits(k2, (M, S), jnp.uint32).T.reshape(S, NUM_TILES, CHUNK)
    bits = jnp.stack([hb, lb], 0)
    return bits.transpose(2, 1, 0, 3).reshape(-1)


def _compact_body(scores_hbm, target_hbm, c_hbm, p_hbm, tgt_v, sc_v, idx_v, pv):
    wid = lax.axis_index("c") * 16 + lax.axis_index("s")
    base = wid * CHUNK

    pltpu.sync_copy(target_hbm, tgt_v)
    for q in range(4):
        pltpu.sync_copy(scores_hbm.at[pl.ds(base + q * 128, 128)], sc_v.at[q])

    zero = jnp.zeros((16,), jnp.int32)
    acc_pre = zero
    acc_tot = zero
    for c in range(NUM_TILES):
        s_c = zero
        for k in range(VPC):
            s_c = s_c + tgt_v[pl.ds(c * CHUNK + k * 16, 16)]
        m = (c < wid).astype(jnp.int32)
        acc_pre = acc_pre + s_c * m
        acc_tot = acc_tot + s_c
    pos_before = jnp.sum(acc_pre)
    p_total = jnp.sum(acc_tot)

    iot = lax.iota(jnp.int32, 16)
    carry = jnp.int32(0)
    for k in range(VPC):
        t = tgt_v[pl.ds(base + k * 16, 16)]
        csum = plsc.cumsum(t)
        excl = csum - t
        prank = pos_before + carry + excl
        gidx = base + k * 16 + iot
        dest = jnp.where(t == 1, prank, M + gidx - prank)
        idx_v[k // 8, pl.ds((k % 8) * 16, 16)] = dest
        carry = carry + jnp.sum(t)

    for q in range(4):
        pltpu.sync_copy(sc_v.at[q], c_hbm.at[idx_v.at[q]])

    @pl.when(wid == 0)
    def _():
        pv[...] = jnp.broadcast_to(p_total, (16,))
        pltpu.sync_copy(pv, p_hbm)


def _sample_body(c_hbm, p_hbm, bits_hbm, out_hbm,
                 negv, posv, bitv, pvv, accv, sem):
    wid = lax.axis_index("c") * 16 + lax.axis_index("s")
    base = wid * CHUNK

    neg_copy = pltpu.async_copy(c_hbm.at[pl.ds(M, M)], negv, sem)
    pltpu.sync_copy(bits_hbm.at[pl.ds(wid * BITS_PER_TILE, BITS_PER_TILE)], bitv)
    pltpu.sync_copy(c_hbm.at[pl.ds(base, CHUNK)], posv)
    pltpu.sync_copy(p_hbm, pvv)

    P = pvv[...]
    s_i = jnp.maximum(M - P, 1)
    s_u = plsc.bitcast(s_i, jnp.uint32)
    s_f = s_i.astype(jnp.float32)
    rcp = 1.0 / s_f
    m1 = lax.rem(jnp.full((16,), 65536, jnp.uint32), s_u)
    m2 = lax.rem(m1 * m1, s_u)
    a3 = lax.rem(m2 * m1, s_u)
    off = s_i * (2 + lax.div(1024 + s_i - 1, s_i))
    lim = jnp.full((16,), 2.0e9, jnp.float32)
    big = jnp.full((16,), 4294967296.0, jnp.float32)
    mask16 = jnp.full((16,), 0xFFFF, jnp.uint32)
    is_one = s_i == 1
    iot = lax.iota(jnp.int32, 16)

    neg_copy.wait()

    def body(i, acc):
        sc = i // VPC
        k = i - sc * VPC
        hb = bitv[pl.ds(sc * 2 * CHUNK + k * 16, 16)]
        lb = bitv[pl.ds(sc * 2 * CHUNK + CHUNK + k * 16, 16)]
        h1 = lax.shift_right_logical(hb, jnp.uint32(16))
        h0 = hb & mask16
        l1 = lax.shift_right_logical(lb, jnp.uint32(16))
        l0 = lb & mask16
        v = h1 * a3 + h0 * m2 + l1 * m1 + l0
        vi = plsc.bitcast(v, jnp.int32)
        vf = vi.astype(jnp.float32)
        vf = jnp.where(vi < 0, vf + big, vf)
        q1 = jnp.minimum(vf * rcp, lim).astype(jnp.int32)
        r1u = v - plsc.bitcast(q1, jnp.uint32) * s_u
        r1 = plsc.bitcast(r1u, jnp.int32) + off
        q2 = (r1.astype(jnp.float32) * rcp).astype(jnp.int32)
        r2 = r1 - q2 * s_i
        r2 = jnp.where(r2 < 0, r2 + s_i, r2)
        r2 = jnp.where(r2 >= s_i, r2 - s_i, r2)
        samp = jnp.where(is_one, 0, r2)
        neg = plsc.load_gather(negv, [samp])
        p = posv[pl.ds(k * 16, 16)]
        rowid = base + k * 16 + iot
        term = jnp.maximum(MARGIN - p + neg, 0.0)
        return acc + jnp.where(rowid < P, term, 0.0)

    acc = lax.fori_loop(0, S * VPC, body, jnp.zeros((16,), jnp.float32))
    accv[...] = acc
    pltpu.sync_copy(accv, out_hbm.at[pl.ds(wid * 16, 16)])


_compact = pl.kernel(
    _compact_body,
    out_type=[jax.ShapeDtypeStruct((2 * M,), jnp.float32),
              jax.ShapeDtypeStruct((16,), jnp.int32)],
    mesh=_mesh,
    compiler_params=pltpu.CompilerParams(needs_layout_passes=False),
    scratch_types=[
        pltpu.VMEM((M,), jnp.int32),
        pltpu.VMEM((4, 128), jnp.float32),
        pltpu.VMEM((4, 128), jnp.int32),
        pltpu.VMEM((16,), jnp.int32),
    ],
)

_sample = pl.kernel(
    _sample_body,
    out_type=jax.ShapeDtypeStruct((NUM_TILES * 16,), jnp.float32),
    mesh=_mesh,
    compiler_params=pltpu.CompilerParams(needs_layout_passes=False),
    scratch_types=[
        pltpu.VMEM((M,), jnp.float32),
        pltpu.VMEM((CHUNK,), jnp.float32),
        pltpu.VMEM((BITS_PER_TILE,), jnp.uint32),
        pltpu.VMEM((16,), jnp.int32),
        pltpu.VMEM((16,), jnp.float32),
        pltpu.SemaphoreType.DMA,
    ],
)


def kernel(scores, target):
    bits = _randint_bits()
    c, pvec = _compact(scores, target)
    parts = _sample(c, pvec, bits)
    P = pvec[0]
    return parts.sum() / (P * S)


# speedup vs baseline: 14.8690x; 1.9765x over previous
import jax
import jax.numpy as jnp
import numpy as np
from jax import lax
from jax.experimental import pallas as pl
from jax.experimental.pallas import tpu as pltpu
from jax.experimental.pallas import tpu_sc as plsc

M = 16384
S = 5
NUM_CORES = 2
NUM_SUBCORES = 16
NUM_TILES = NUM_CORES * NUM_SUBCORES
CHUNK = M // NUM_TILES
CCHUNK = M // NUM_SUBCORES
CVPC = CCHUNK // 16
BITS_PER_TILE = 2 * S * CHUNK
MARGIN = 1.0

_mesh = plsc.VectorSubcoreMesh(core_axis_name="c", subcore_axis_name="s")


def _randint_bits():
    k1, k2 = jax.random.split(jax.random.key(42))
    hb = jax.random.bits(k1, (M, S), jnp.uint32).T.reshape(S, NUM_TILES, CHUNK)
    lb = jax.random.bits(k2, (M, S), jnp.uint32).T.reshape(S, NUM_TILES, CHUNK)
    bits = jnp.stack([hb, lb], 0)
    return bits.transpose(2, 1, 0, 3).reshape(-1)


def _body(scores_hbm, target_hbm, bits_hbm, parts_hbm, p_hbm, bounce_hbm,
          tgt_v, sc_v, idx_v, cshared, negv, posv, bitv, pv, accv, sem):
    cid = lax.axis_index("c")
    sid = lax.axis_index("s")
    wid = cid * NUM_SUBCORES + sid
    cbase = sid * CCHUNK
    rbase = wid * CHUNK

    pltpu.sync_copy(target_hbm, tgt_v)
    for q in range(8):
        pltpu.sync_copy(scores_hbm.at[pl.ds(cbase + q * 128, 128)], sc_v.at[q])

    zero = jnp.zeros((16,), jnp.int32)
    acc_pre = zero
    acc_tot = zero
    for c in range(NUM_SUBCORES):
        s_c = zero
        for k in range(CVPC):
            s_c = s_c + tgt_v[pl.ds(c * CCHUNK + k * 16, 16)]
        m = (c < sid).astype(jnp.int32)
        acc_pre = acc_pre + s_c * m
        acc_tot = acc_tot + s_c
    pos_before = jnp.sum(acc_pre)
    p_total = jnp.sum(acc_tot)

    iot = lax.iota(jnp.int32, 16)
    carry = jnp.int32(0)
    for k in range(CVPC):
        t = tgt_v[pl.ds(cbase + k * 16, 16)]
        csum = plsc.cumsum(t)
        excl = csum - t
        prank = pos_before + carry + excl
        gidx = cbase + k * 16 + iot
        dest = jnp.where(t == 1, prank, M + gidx - prank)
        idx_v[k // 8, pl.ds((k % 8) * 16, 16)] = dest
        carry = carry + jnp.sum(t)

    for q in range(8):
        pltpu.sync_copy(sc_v.at[q], cshared.at[idx_v.at[q]])

    plsc.subcore_barrier()

    out_off = cid * 2 * M + sid * (2 * M // NUM_SUBCORES)
    pltpu.sync_copy(cshared.at[pl.ds(sid * (2 * M // NUM_SUBCORES),
                                     2 * M // NUM_SUBCORES)],
                    bounce_hbm.at[pl.ds(out_off, 2 * M // NUM_SUBCORES)])

    plsc.subcore_barrier()

    neg_copy = pltpu.async_copy(bounce_hbm.at[pl.ds(cid * 2 * M + M, M)],
                                negv, sem)
    pltpu.sync_copy(bits_hbm.at[pl.ds(wid * BITS_PER_TILE, BITS_PER_TILE)], bitv)
    pltpu.sync_copy(bounce_hbm.at[pl.ds(cid * 2 * M + rbase, CHUNK)], posv)

    P = jnp.broadcast_to(p_total, (16,))
    s_i = jnp.maximum(M - P, 1)
    s_u = plsc.bitcast(s_i, jnp.uint32)
    s_f = s_i.astype(jnp.float32)
    rcp = 1.0 / s_f
    m1 = lax.rem(jnp.full((16,), 65536, jnp.uint32), s_u)
    m2 = lax.rem(m1 * m1, s_u)
    a3 = lax.rem(m2 * m1, s_u)
    off = s_i * (2 + lax.div(1024 + s_i - 1, s_i))
    lim = jnp.full((16,), 2.0e9, jnp.float32)
    big = jnp.full((16,), 4294967296.0, jnp.float32)
    mask16 = jnp.full((16,), 0xFFFF, jnp.uint32)
    is_one = s_i == 1

    neg_copy.wait()

    def body(i, acc):
        sc = i // (CHUNK // 16)
        k = i - sc * (CHUNK // 16)
        hb = bitv[pl.ds(sc * 2 * CHUNK + k * 16, 16)]
        lb = bitv[pl.ds(sc * 2 * CHUNK + CHUNK + k * 16, 16)]
        h1 = lax.shift_right_logical(hb, jnp.uint32(16))
        h0 = hb & mask16
        l1 = lax.shift_right_logical(lb, jnp.uint32(16))
        l0 = lb & mask16
        v = h1 * a3 + h0 * m2 + l1 * m1 + l0
        vi = plsc.bitcast(v, jnp.int32)
        vf = vi.astype(jnp.float32)
        vf = jnp.where(vi < 0, vf + big, vf)
        q1 = jnp.minimum(vf * rcp, lim).astype(jnp.int32)
        r1u = v - plsc.bitcast(q1, jnp.uint32) * s_u
        r1 = plsc.bitcast(r1u, jnp.int32) + off
        q2 = (r1.astype(jnp.float32) * rcp).astype(jnp.int32)
        r2 = r1 - q2 * s_i
        r2 = jnp.where(r2 < 0, r2 + s_i, r2)
        r2 = jnp.where(r2 >= s_i, r2 - s_i, r2)
        samp = jnp.where(is_one, 0, r2)
        neg = plsc.load_gather(negv, [samp])
        p = posv[pl.ds(k * 16, 16)]
        rowid = rbase + k * 16 + iot
        term = jnp.maximum(MARGIN - p + neg, 0.0)
        return acc + jnp.where(rowid < P, term, 0.0)

    acc = lax.fori_loop(0, S * (CHUNK // 16), body, jnp.zeros((16,), jnp.float32))
    accv[...] = acc
    pltpu.sync_copy(accv, parts_hbm.at[pl.ds(wid * 16, 16)])

    @pl.when(wid == 0)
    def _():
        pv[...] = P
        pltpu.sync_copy(pv, p_hbm)


_fused = pl.kernel(
    _body,
    out_type=[jax.ShapeDtypeStruct((NUM_TILES * 16,), jnp.float32),
              jax.ShapeDtypeStruct((16,), jnp.int32),
              jax.ShapeDtypeStruct((NUM_CORES * 2 * M,), jnp.float32)],
    mesh=_mesh,
    compiler_params=pltpu.CompilerParams(needs_layout_passes=False),
    scratch_types=[
        pltpu.VMEM((M,), jnp.int32),
        pltpu.VMEM((8, 128), jnp.float32),
        pltpu.VMEM((8, 128), jnp.int32),
        pltpu.VMEM_SHARED((2 * M,), jnp.float32),
        pltpu.VMEM((M,), jnp.float32),
        pltpu.VMEM((CHUNK,), jnp.float32),
        pltpu.VMEM((BITS_PER_TILE,), jnp.uint32),
        pltpu.VMEM((16,), jnp.int32),
        pltpu.VMEM((16,), jnp.float32),
        pltpu.SemaphoreType.DMA,
    ],
)


def kernel(scores, target):
    bits = _randint_bits()
    parts, pvec, _ = _fused(scores, target, bits)
    P = pvec[0]
    return parts.sum() / (P * S)


# speedup vs baseline: 19.4151x; 1.3057x over previous
import jax
import jax.numpy as jnp
import numpy as np
from jax import lax
from jax.experimental import pallas as pl
from jax.experimental.pallas import tpu as pltpu
from jax.experimental.pallas import tpu_sc as plsc

M = 16384
S = 5
NUM_CORES = 2
NUM_SUBCORES = 16
NUM_TILES = NUM_CORES * NUM_SUBCORES
CHUNK = M // NUM_TILES
CCHUNK = M // NUM_SUBCORES
CVPC = CCHUNK // 16
BITS_PER_TILE = 2 * S * CHUNK
MARGIN = 1.0

_mesh = plsc.VectorSubcoreMesh(core_axis_name="c", subcore_axis_name="s")


def _tf_hash(k1, k2, c1, c2):
    k1 = np.uint32(k1)
    k2 = np.uint32(k2)
    ks = [k1, k2, np.uint32(k1 ^ k2 ^ np.uint32(0x1BD11BDA))]
    rot = [(13, 15, 26, 6), (17, 29, 16, 24)]
    x0 = (np.asarray(c1, np.uint32) + k1).astype(np.uint32)
    x1 = (np.asarray(c2, np.uint32) + k2).astype(np.uint32)
    for g in range(5):
        for r in rot[g % 2]:
            x0 = (x0 + x1).astype(np.uint32)
            x1 = ((x1 << np.uint32(r)) | (x1 >> np.uint32(32 - r))).astype(np.uint32)
            x1 = (x1 ^ x0).astype(np.uint32)
        x0 = (x0 + ks[(g + 1) % 3]).astype(np.uint32)
        x1 = (x1 + ks[(g + 2) % 3] + np.uint32(g + 1)).astype(np.uint32)
    return x0, x1


def _randint_bits_np():
    base = np.array([0, 42], dtype=np.uint32)
    s0, s1 = _tf_hash(base[0], base[1],
                      np.zeros(2, np.uint32), np.arange(2, dtype=np.uint32))
    n = M * S
    zeros = np.zeros(n, np.uint32)
    iota = np.arange(n, dtype=np.uint32)
    hx0, hx1 = _tf_hash(s0[0], s1[0], zeros, iota)
    lx0, lx1 = _tf_hash(s0[1], s1[1], zeros, iota)
    hb = (hx0 ^ hx1).reshape(M, S).T.reshape(S, NUM_TILES, CHUNK)
    lb = (lx0 ^ lx1).reshape(M, S).T.reshape(S, NUM_TILES, CHUNK)
    bits = np.stack([hb, lb], 0)
    return np.ascontiguousarray(bits.transpose(2, 1, 0, 3)).reshape(-1)


_BITS = _randint_bits_np()


def _body(scores_hbm, target_hbm, bits_hbm, parts_hbm, p_hbm, bounce_hbm,
          tgt_v, sc_v, idx_v, cshared, negv, posv, bitv, pv, accv,
          sem0, sem1, sem2):
    cid = lax.axis_index("c")
    sid = lax.axis_index("s")
    wid = cid * NUM_SUBCORES + sid
    cbase = sid * CCHUNK
    rbase = wid * CHUNK

    tgt_d = pltpu.async_copy(target_hbm, tgt_v, sem0)
    bits_d = pltpu.async_copy(
        bits_hbm.at[pl.ds(wid * BITS_PER_TILE, BITS_PER_TILE)], bitv, sem1)
    sc_d = [pltpu.async_copy(scores_hbm.at[pl.ds(cbase + q * 128, 128)],
                             sc_v.at[q], sem2)
            for q in range(8)]
    tgt_d.wait()

    zero = jnp.zeros((16,), jnp.int32)
    acc_pre = zero
    acc_tot = zero
    for c in range(NUM_SUBCORES):
        s_c = zero
        for k in range(CVPC):
            s_c = s_c + tgt_v[pl.ds(c * CCHUNK + k * 16, 16)]
        m = (c < sid).astype(jnp.int32)
        acc_pre = acc_pre + s_c * m
        acc_tot = acc_tot + s_c
    pos_before = jnp.sum(acc_pre)
    p_total = jnp.sum(acc_tot)

    iot = lax.iota(jnp.int32, 16)
    carry = jnp.int32(0)
    for k in range(CVPC):
        t = tgt_v[pl.ds(cbase + k * 16, 16)]
        csum = plsc.cumsum(t)
        excl = csum - t
        prank = pos_before + carry + excl
        gidx = cbase + k * 16 + iot
        dest = jnp.where(t == 1, prank, M + gidx - prank)
        idx_v[k // 8, pl.ds((k % 8) * 16, 16)] = dest
        carry = carry + csum[15]

    for d in sc_d:
        d.wait()
    scat_d = [pltpu.async_copy(sc_v.at[q], cshared.at[idx_v.at[q]], sem2)
              for q in range(8)]
    for d in scat_d:
        d.wait()

    plsc.subcore_barrier()

    out_off = cid * 2 * M + sid * (2 * M // NUM_SUBCORES)
    pltpu.sync_copy(cshared.at[pl.ds(sid * (2 * M // NUM_SUBCORES),
                                     2 * M // NUM_SUBCORES)],
                    bounce_hbm.at[pl.ds(out_off, 2 * M // NUM_SUBCORES)])

    plsc.subcore_barrier()

    neg_copy = pltpu.async_copy(bounce_hbm.at[pl.ds(cid * 2 * M + M, M)],
                                negv, sem0)
    pos_copy = pltpu.async_copy(bounce_hbm.at[pl.ds(cid * 2 * M + rbase, CHUNK)],
                                posv, sem2)

    P = jnp.broadcast_to(p_total, (16,))
    s_i = jnp.maximum(M - P, 1)
    s_u = plsc.bitcast(s_i, jnp.uint32)
    s_f = s_i.astype(jnp.float32)
    rcp = 1.0 / s_f
    m1 = lax.rem(jnp.full((16,), 65536, jnp.uint32), s_u)
    m2 = lax.rem(m1 * m1, s_u)
    a3 = lax.rem(m2 * m1, s_u)
    off = s_i * (2 + lax.div(1024 + s_i - 1, s_i))
    lim = jnp.full((16,), 2.0e9, jnp.float32)
    big = jnp.full((16,), 4294967296.0, jnp.float32)
    mask16 = jnp.full((16,), 0xFFFF, jnp.uint32)
    is_one = s_i == 1

    bits_d.wait()
    pos_copy.wait()
    neg_copy.wait()

    def body(i, acc):
        sc = i // (CHUNK // 16)
        k = i - sc * (CHUNK // 16)
        hb = bitv[pl.ds(sc * 2 * CHUNK + k * 16, 16)]
        lb = bitv[pl.ds(sc * 2 * CHUNK + CHUNK + k * 16, 16)]
        h1 = lax.shift_right_logical(hb, jnp.uint32(16))
        h0 = hb & mask16
        l1 = lax.shift_right_logical(lb, jnp.uint32(16))
        l0 = lb & mask16
        v = h1 * a3 + h0 * m2 + l1 * m1 + l0
        vi = plsc.bitcast(v, jnp.int32)
        vf = vi.astype(jnp.float32)
        vf = jnp.where(vi < 0, vf + big, vf)
        q1 = jnp.minimum(vf * rcp, lim).astype(jnp.int32)
        r1u = v - plsc.bitcast(q1, jnp.uint32) * s_u
        r1 = plsc.bitcast(r1u, jnp.int32) + off
        q2 = (r1.astype(jnp.float32) * rcp).astype(jnp.int32)
        r2 = r1 - q2 * s_i
        r2 = jnp.where(r2 < 0, r2 + s_i, r2)
        r2 = jnp.where(r2 >= s_i, r2 - s_i, r2)
        samp = jnp.where(is_one, 0, r2)
        neg = plsc.load_gather(negv, [samp])
        p = posv[pl.ds(k * 16, 16)]
        rowid = rbase + k * 16 + iot
        term = jnp.maximum(MARGIN - p + neg, 0.0)
        return acc + jnp.where(rowid < P, term, 0.0)

    acc = lax.fori_loop(0, S * (CHUNK // 16), body, jnp.zeros((16,), jnp.float32))
    accv[...] = acc
    pltpu.sync_copy(accv, parts_hbm.at[pl.ds(wid * 16, 16)])

    @pl.when(wid == 0)
    def _():
        pv[...] = P
        pltpu.sync_copy(pv, p_hbm)


_fused = pl.kernel(
    _body,
    out_type=[jax.ShapeDtypeStruct((NUM_TILES * 16,), jnp.float32),
              jax.ShapeDtypeStruct((16,), jnp.int32),
              jax.ShapeDtypeStruct((NUM_CORES * 2 * M,), jnp.float32)],
    mesh=_mesh,
    compiler_params=pltpu.CompilerParams(needs_layout_passes=False),
    scratch_types=[
        pltpu.VMEM((M,), jnp.int32),
        pltpu.VMEM((8, 128), jnp.float32),
        pltpu.VMEM((8, 128), jnp.int32),
        pltpu.VMEM_SHARED((2 * M,), jnp.float32),
        pltpu.VMEM((M,), jnp.float32),
        pltpu.VMEM((CHUNK,), jnp.float32),
        pltpu.VMEM((BITS_PER_TILE,), jnp.uint32),
        pltpu.VMEM((16,), jnp.int32),
        pltpu.VMEM((16,), jnp.float32),
        pltpu.SemaphoreType.DMA,
        pltpu.SemaphoreType.DMA,
        pltpu.SemaphoreType.DMA,
    ],
)


def kernel(scores, target):
    bits = jnp.asarray(_BITS)
    parts, pvec, _ = _fused(scores, target, bits)
    P = pvec[0]
    return parts.sum() / (P * S)
